# baseline (speedup 1.0000x reference)
import functools

import jax
import jax.numpy as jnp
from jax import lax
from jax.experimental import pallas as pl
from jax.experimental.pallas import tpu as pltpu
from jax.experimental.pallas import tpu_sc as plsc

N = 10000
G = 20
PER = 500
D1, D2, D3 = 256, 256, 128
K = 250
MLP1_OUT = 512

NP = 10240
NCHUNK = NP // 16
EP = 163840
ET = EP // 32
ECH = ET // 128


def _tc_front_body(bp_ref, cc_ref, mf_ref, Wbp_ref, bbp_ref, Wbp1_ref,
                   bbp1_ref, Watt_ref, batt_ref, vatt_ref, Wsc_ref, bsc_ref,
                   fun_ref, h_ref):
    Wbp = Wbp_ref[...]
    bbp = bbp_ref[...]
    Wbp1 = Wbp1_ref[...]
    bbp1 = bbp1_ref[...]
    Watt = Watt_ref[...]
    batt = batt_ref[...]
    vatt = vatt_ref[...]

    bf = lambda x: x.astype(jnp.bfloat16)

    def dot(a, b):
        return jnp.dot(bf(a), bf(b), preferred_element_type=jnp.float32)

    def proj(x):
        h1 = jnp.maximum(dot(x, Wbp) + bbp, 0.0)
        return jnp.maximum(dot(h1, Wbp1) + bbp1, 0.0)

    def att_logit(h2):
        t = jnp.tanh(dot(h2, Watt) + batt)
        return dot(t, vatt)

    h_bp = proj(bp_ref[...])
    h_cc = proj(cc_ref[...])
    h_mf = proj(mf_ref[...])

    e0 = att_logit(h_bp)
    e1 = att_logit(h_cc)
    e2 = att_logit(h_mf)
    m = jnp.maximum(jnp.maximum(e0, e1), e2)
    u0 = jnp.exp(e0 - m)
    u1 = jnp.exp(e1 - m)
    u2 = jnp.exp(e2 - m)
    denom = u0 + u1 + u2
    a0 = u0 / denom
    a1 = u1 / denom
    a2 = u2 / denom
    fun = a0 * h_bp + a1 * h_cc + a2 * h_mf
    fun_ref[...] = fun
    h_ref[...] = dot(fun, Wsc_ref[...]) + bsc_ref[...]


def _tc_front(Fea_BP, fea_CC, fea_MF, W_bp, b_bp, W_bp1, b_bp1, W_att, b_att,
              v_att, W_score, b_score):
    BR = 1000
    full = lambda shape: pl.BlockSpec(shape, lambda i: (0, 0))
    return pl.pallas_call(
        _tc_front_body,
        grid=(N // BR,),
        in_specs=[
            pl.BlockSpec((BR, D1), lambda i: (i, 0)),
            pl.BlockSpec((BR, D1), lambda i: (i, 0)),
            pl.BlockSpec((BR, D1), lambda i: (i, 0)),
            full((D1, D2)), full((1, D2)),
            full((D2, D3)), full((1, D3)),
            full((D3, D3)), full((1, D3)), full((D3, 1)),
            full((D3, 1)), full((1, 1)),
        ],
        out_specs=[
            pl.BlockSpec((BR, D3), lambda i: (i, 0)),
            pl.BlockSpec((BR, 1), lambda i: (i, 0)),
        ],
        out_shape=[
            jax.ShapeDtypeStruct((N, D3), jnp.float32),
            jax.ShapeDtypeStruct((N, 1), jnp.float32),
        ],
    )(Fea_BP.astype(jnp.bfloat16), fea_CC.astype(jnp.bfloat16),
      fea_MF.astype(jnp.bfloat16), W_bp.astype(jnp.bfloat16),
      b_bp.reshape(1, D2), W_bp1.astype(jnp.bfloat16),
      b_bp1.reshape(1, D3), W_att.astype(jnp.bfloat16),
      b_att.reshape(1, D3), v_att.reshape(D3, 1).astype(jnp.bfloat16),
      W_score.astype(jnp.bfloat16), b_score.reshape(1, 1))


PMAX = 16384
TABN = PMAX + 128


def _tc_mid_body(d0_ref, d1_ref, h_ref, hi_ref, di_ref):
    deg = d0_ref[...] + d1_ref[...] + 1.0
    deg = jnp.maximum(deg, 1.0)
    hi_ref[...] = h_ref[...] * (1.0 / deg)
    di_ref[...] = deg.astype(jnp.int32)


def _tc_mid(deg_p, h_pad):
    full = pl.BlockSpec((NP // 128, 128), lambda: (0, 0))
    return pl.pallas_call(
        _tc_mid_body,
        in_specs=[full, full, full],
        out_specs=[full, full],
        out_shape=[
            jax.ShapeDtypeStruct((NP // 128, 128), jnp.float32),
            jax.ShapeDtypeStruct((NP // 128, 128), jnp.int32),
        ],
    )(deg_p[0].reshape(NP // 128, 128), deg_p[1].reshape(NP // 128, 128),
      h_pad.reshape(NP // 128, 128))


def _sc_deg_body(dst_hbm, out_hbm, idx_v, ones_v, zeros_v, sem, acc_sh):
    cid = lax.axis_index("c")
    sid = lax.axis_index("s")
    wid = sid * 2 + cid

    zero16 = jnp.zeros((16,), jnp.float32)
    one16 = jnp.ones((16,), jnp.float32)

    def init_body(i, _):
        zeros_v[pl.ds(i * 16, 16)] = zero16
        return 0
    lax.fori_loop(0, NCHUNK // 16, init_body, 0)
    for i in range(8):
        ones_v[pl.ds(i * 16, 16)] = one16

    pltpu.sync_copy(zeros_v, acc_sh.at[pl.ds(sid * NCHUNK, NCHUNK)])
    pltpu.sync_copy(dst_hbm.at[pl.ds(wid * ECH, ECH)], idx_v)
    plsc.subcore_barrier()

    def scat_body(j, _):
        pltpu.async_copy(ones_v, acc_sh.at[idx_v.at[j]], sem, add=True)
        return 0
    lax.fori_loop(0, ECH, scat_body, 0)
    def drain_body(j, _):
        pltpu.make_async_copy(ones_v, acc_sh.at[idx_v.at[j]], sem).wait()
        return 0
    lax.fori_loop(0, ECH, drain_body, 0)
    plsc.subcore_barrier()

    pltpu.sync_copy(acc_sh.at[pl.ds(sid * NCHUNK, NCHUNK)],
                    out_hbm.at[cid, pl.ds(sid * NCHUNK, NCHUNK)])


def _sc_msg_body(src_hbm, dst_hbm, h_hbm, deg_hbm, tab_hbm, out_hbm,
                 idx_v, src_v, val_v, h_v, degi_v, tab_v, zeros_v, sem, acc_sh):
    cid = lax.axis_index("c")
    sid = lax.axis_index("s")
    wid = sid * 2 + cid

    zero16 = jnp.zeros((16,), jnp.float32)

    def init_body(i, _):
        zeros_v[pl.ds(i * 16, 16)] = zero16
        return 0
    lax.fori_loop(0, NCHUNK // 16, init_body, 0)

    pltpu.sync_copy(zeros_v, acc_sh.at[pl.ds(sid * NCHUNK, NCHUNK)])
    pltpu.sync_copy(dst_hbm.at[pl.ds(wid * ECH, ECH)], idx_v)
    pltpu.sync_copy(src_hbm.at[pl.ds(wid * ET, ET)], src_v)
    pltpu.sync_copy(h_hbm, h_v)
    pltpu.sync_copy(deg_hbm, degi_v)
    pltpu.sync_copy(tab_hbm, tab_v)
    plsc.subcore_barrier()

    def chunk_body(j, _):
        for i in range(8):
            t = j * 8 + i
            s_idx = src_v[pl.ds(t * 16, 16)]
            d_idx = idx_v[j, pl.ds(i * 16, 16)]
            hs = plsc.load_gather(h_v, [s_idx])
            dsg = plsc.load_gather(degi_v, [s_idx])
            ddg = plsc.load_gather(degi_v, [d_idx])
            p = jnp.minimum(dsg * ddg, PMAX)
            nrm = plsc.load_gather(tab_v, [p])
            val_v[pl.ds(t * 16, 16)] = hs * nrm
        pltpu.async_copy(val_v.at[pl.ds(j * 128, 128)],
                         acc_sh.at[idx_v.at[j]], sem, add=True)
        return 0
    lax.fori_loop(0, ECH, chunk_body, 0)
    def drain_body(j, _):
        pltpu.make_async_copy(val_v.at[pl.ds(j * 128, 128)],
                              acc_sh.at[idx_v.at[j]], sem).wait()
        return 0
    lax.fori_loop(0, ECH, drain_body, 0)
    plsc.subcore_barrier()

    pltpu.sync_copy(acc_sh.at[pl.ds(sid * NCHUNK, NCHUNK)],
                    out_hbm.at[cid, pl.ds(sid * NCHUNK, NCHUNK)])


def _tc_back_body(s0_ref, s1_ref, hi_ref, s0t_ref, s1t_ref, hit_ref,
                  fun_ref, Wm_ref, bm_ref, lo_ref, fea_ref):
    srow = ((s0_ref[...] + s1_ref[...]) + hi_ref[...])[0]
    scol = ((s0t_ref[...] + s1t_ref[...]) + hit_ref[...])[0]

    ii = lax.broadcasted_iota(jnp.int32, (PER, PER), 0)
    jj = lax.broadcasted_iota(jnp.int32, (PER, PER), 1)
    beats = (scol > srow) | ((scol == srow) & (ii < jj))
    rank = jnp.dot(jnp.ones((1, PER), jnp.bfloat16),
                   beats.astype(jnp.bfloat16),
                   preferred_element_type=jnp.float32)

    pp = lax.broadcasted_iota(jnp.int32, (K, PER), 0).astype(jnp.float32)
    P = (pp == rank).astype(jnp.float32)

    X = fun_ref[...][0] * jnp.tanh(scol)
    fea_st = jnp.dot(P, X, preferred_element_type=jnp.float32,
                     precision=jax.lax.Precision.HIGHEST)
    fea = jnp.dot(fea_st.astype(jnp.bfloat16), Wm_ref[...].astype(jnp.bfloat16),
                  preferred_element_type=jnp.float32) + bm_ref[...]
    fea_ref[...] = fea[None]
    lo_ref[...] = jax.nn.sigmoid(fea)[None]


def _tc_back(s0, s1, hi, Funrep, W_mlp1, b_mlp1):
    row = pl.BlockSpec((1, 1, PER), lambda i: (i, 0, 0))
    col = pl.BlockSpec((1, PER, 1), lambda i: (i, 0, 0))
    full = lambda shape: pl.BlockSpec(shape, lambda i: (0, 0))
    r3 = lambda x: x.reshape(G, 1, PER)
    c3 = lambda x: x.reshape(G, PER, 1)
    return pl.pallas_call(
        _tc_back_body,
        grid=(G,),
        in_specs=[row, row, row, col, col, col,
                  pl.BlockSpec((1, PER, D3), lambda i: (i, 0, 0)),
                  full((D3, MLP1_OUT)), full((1, MLP1_OUT))],
        out_specs=[pl.BlockSpec((1, K, MLP1_OUT), lambda i: (i, 0, 0)),
                   pl.BlockSpec((1, K, MLP1_OUT), lambda i: (i, 0, 0))],
        out_shape=[
            jax.ShapeDtypeStruct((G, K, MLP1_OUT), jnp.float32),
            jax.ShapeDtypeStruct((G, K, MLP1_OUT), jnp.float32),
        ],
    )(r3(s0), r3(s1), r3(hi), c3(s0), c3(s1), c3(hi),
      Funrep.reshape(G, PER, D3), W_mlp1, b_mlp1.reshape(1, MLP1_OUT))


@functools.lru_cache(maxsize=None)
def _sc_kernels():
    mesh = plsc.VectorSubcoreMesh(core_axis_name="c", subcore_axis_name="s")
    params = pltpu.CompilerParams(needs_layout_passes=False)
    sc_deg = pl.kernel(
        _sc_deg_body, mesh=mesh, compiler_params=params,
        out_type=jax.ShapeDtypeStruct((2, NP), jnp.float32),
        scratch_types=[
            pltpu.VMEM((ECH, 128), jnp.int32),
            pltpu.VMEM((128,), jnp.float32),
            pltpu.VMEM((NCHUNK,), jnp.float32),
            pltpu.SemaphoreType.DMA,
            pltpu.VMEM_SHARED((NP,), jnp.float32),
        ],
    )
    sc_msg = pl.kernel(
        _sc_msg_body, mesh=mesh, compiler_params=params,
        out_type=jax.ShapeDtypeStruct((2, NP), jnp.float32),
        scratch_types=[
            pltpu.VMEM((ECH, 128), jnp.int32),
            pltpu.VMEM((ET,), jnp.int32),
            pltpu.VMEM((ET,), jnp.float32),
            pltpu.VMEM((NP,), jnp.float32),
            pltpu.VMEM((NP,), jnp.int32),
            pltpu.VMEM((TABN,), jnp.float32),
            pltpu.VMEM((NCHUNK,), jnp.float32),
            pltpu.SemaphoreType.DMA,
            pltpu.VMEM_SHARED((NP,), jnp.float32),
        ],
    )
    return sc_deg, sc_msg


def kernel(Fea_BP, fea_CC, fea_MF, edge_index, batch, W_bp, b_bp, W_bp1,
           b_bp1, W_att, b_att, v_att, W_score, b_score, W_mlp1, b_mlp1):
    Funrep, h = _tc_front(Fea_BP, fea_CC, fea_MF, W_bp, b_bp, W_bp1, b_bp1,
                          W_att, b_att, v_att, W_score, b_score)

    E_ = edge_index.shape[1]
    fill = jnp.full((EP - E_,), N, dtype=jnp.int32)
    src = jnp.concatenate([edge_index[0], fill])
    dst2d = jnp.concatenate([edge_index[1], fill]).reshape(EP // 128, 128)

    sc_deg, sc_msg = _sc_kernels()
    deg_p = sc_deg(dst2d)
    h_pad = jnp.pad(h[:, 0], (0, NP - N))
    hi, deg_i = _tc_mid(deg_p, h_pad)
    tab = 1.0 / jnp.sqrt(jnp.maximum(jnp.arange(TABN, dtype=jnp.float32), 1.0))
    S_p = sc_msg(src, dst2d, h_pad, deg_i.reshape(NP), tab)

    s0 = S_p[0, :N].reshape(G, PER)
    s1 = S_p[1, :N].reshape(G, PER)
    hi_n = hi.reshape(NP)[:N].reshape(G, PER)

    lo, fea_LO = _tc_back(s0, s1, hi_n, Funrep, W_mlp1, b_mlp1)
    return (lo.reshape(G * K, MLP1_OUT), fea_LO.reshape(G * K, MLP1_OUT), Funrep)


# speedup vs baseline: 1.1184x; 1.1184x over previous
import functools

import jax
import jax.numpy as jnp
from jax import lax
from jax.experimental import pallas as pl
from jax.experimental.pallas import tpu as pltpu
from jax.experimental.pallas import tpu_sc as plsc

N = 10000
G = 20
PER = 500
D1, D2, D3 = 256, 256, 128
K = 250
MLP1_OUT = 512

NP = 10240
NCHUNK = NP // 16
EP = 163840
ET = EP // 32
ECH = ET // 128


def _tc_front_body(bp_ref, cc_ref, mf_ref, Wbp_ref, bbp_ref, Wbp1_ref,
                   bbp1_ref, Watt_ref, batt_ref, vatt_ref, Wsc_ref, bsc_ref,
                   fun_ref, h_ref):
    Wbp = Wbp_ref[...]
    bbp = bbp_ref[...]
    Wbp1 = Wbp1_ref[...]
    bbp1 = bbp1_ref[...]
    Watt = Watt_ref[...]
    batt = batt_ref[...]
    vatt = vatt_ref[...]

    bf = lambda x: x.astype(jnp.bfloat16)

    def dot(a, b):
        return jnp.dot(bf(a), bf(b), preferred_element_type=jnp.float32)

    def proj(x):
        h1 = jnp.maximum(dot(x, Wbp) + bbp, 0.0)
        return jnp.maximum(dot(h1, Wbp1) + bbp1, 0.0)

    def att_logit(h2):
        t = jnp.tanh(dot(h2, Watt) + batt)
        return dot(t, vatt)

    h_bp = proj(bp_ref[...])
    h_cc = proj(cc_ref[...])
    h_mf = proj(mf_ref[...])

    e0 = att_logit(h_bp)
    e1 = att_logit(h_cc)
    e2 = att_logit(h_mf)
    m = jnp.maximum(jnp.maximum(e0, e1), e2)
    u0 = jnp.exp(e0 - m)
    u1 = jnp.exp(e1 - m)
    u2 = jnp.exp(e2 - m)
    denom = u0 + u1 + u2
    a0 = u0 / denom
    a1 = u1 / denom
    a2 = u2 / denom
    fun = a0 * h_bp + a1 * h_cc + a2 * h_mf
    fun_ref[...] = fun
    h_ref[...] = dot(fun, Wsc_ref[...]) + bsc_ref[...]


def _tc_front(Fea_BP, fea_CC, fea_MF, W_bp, b_bp, W_bp1, b_bp1, W_att, b_att,
              v_att, W_score, b_score):
    BR = 1000
    full = lambda shape: pl.BlockSpec(shape, lambda i: (0, 0))
    return pl.pallas_call(
        _tc_front_body,
        grid=(N // BR,),
        in_specs=[
            pl.BlockSpec((BR, D1), lambda i: (i, 0)),
            pl.BlockSpec((BR, D1), lambda i: (i, 0)),
            pl.BlockSpec((BR, D1), lambda i: (i, 0)),
            full((D1, D2)), full((1, D2)),
            full((D2, D3)), full((1, D3)),
            full((D3, D3)), full((1, D3)), full((D3, 1)),
            full((D3, 1)), full((1, 1)),
        ],
        out_specs=[
            pl.BlockSpec((BR, D3), lambda i: (i, 0)),
            pl.BlockSpec((BR, 1), lambda i: (i, 0)),
        ],
        out_shape=[
            jax.ShapeDtypeStruct((N, D3), jnp.float32),
            jax.ShapeDtypeStruct((N, 1), jnp.float32),
        ],
    )(Fea_BP, fea_CC, fea_MF, W_bp, b_bp.reshape(1, D2), W_bp1,
      b_bp1.reshape(1, D3), W_att, b_att.reshape(1, D3), v_att.reshape(D3, 1),
      W_score, b_score.reshape(1, 1))


PMAX = 16384
TABN = PMAX + 128


def _tc_mid_body(d0_ref, d1_ref, h_ref, hi_ref, di_ref):
    deg = d0_ref[...] + d1_ref[...] + 1.0
    deg = jnp.maximum(deg, 1.0)
    hi_ref[...] = h_ref[...] * (1.0 / deg)
    di_ref[...] = deg.astype(jnp.int32)


def _tc_mid(deg_p, h_pad):
    full = pl.BlockSpec((NP // 128, 128), lambda: (0, 0))
    return pl.pallas_call(
        _tc_mid_body,
        in_specs=[full, full, full],
        out_specs=[full, full],
        out_shape=[
            jax.ShapeDtypeStruct((NP // 128, 128), jnp.float32),
            jax.ShapeDtypeStruct((NP // 128, 128), jnp.int32),
        ],
    )(deg_p[0].reshape(NP // 128, 128), deg_p[1].reshape(NP // 128, 128),
      h_pad.reshape(NP // 128, 128))


def _sc_deg_body(dst_hbm, out_hbm, idx_v, ones_v, zeros_v, sem, acc_sh):
    cid = lax.axis_index("c")
    sid = lax.axis_index("s")
    wid = sid * 2 + cid

    zero16 = jnp.zeros((16,), jnp.float32)
    one16 = jnp.ones((16,), jnp.float32)

    def init_body(i, _):
        zeros_v[pl.ds(i * 16, 16)] = zero16
        return 0
    lax.fori_loop(0, NCHUNK // 16, init_body, 0)
    for i in range(8):
        ones_v[pl.ds(i * 16, 16)] = one16

    pltpu.sync_copy(zeros_v, acc_sh.at[pl.ds(sid * NCHUNK, NCHUNK)])
    pltpu.sync_copy(dst_hbm.at[pl.ds(wid * ECH, ECH)], idx_v)
    plsc.subcore_barrier()

    def scat_body(j, _):
        pltpu.async_copy(ones_v, acc_sh.at[idx_v.at[j]], sem, add=True)
        return 0
    lax.fori_loop(0, ECH, scat_body, 0)
    def drain_body(j, _):
        pltpu.make_async_copy(ones_v, acc_sh.at[idx_v.at[j]], sem).wait()
        return 0
    lax.fori_loop(0, ECH, drain_body, 0)
    plsc.subcore_barrier()

    pltpu.sync_copy(acc_sh.at[pl.ds(sid * NCHUNK, NCHUNK)],
                    out_hbm.at[cid, pl.ds(sid * NCHUNK, NCHUNK)])


def _sc_msg_body(src_hbm, dst_hbm, h_hbm, deg_hbm, tab_hbm, out_hbm,
                 idx_v, src_v, val_v, h_v, degi_v, tab_v, zeros_v, sem, acc_sh):
    cid = lax.axis_index("c")
    sid = lax.axis_index("s")
    wid = sid * 2 + cid

    zero16 = jnp.zeros((16,), jnp.float32)

    def init_body(i, _):
        zeros_v[pl.ds(i * 16, 16)] = zero16
        return 0
    lax.fori_loop(0, NCHUNK // 16, init_body, 0)

    pltpu.sync_copy(zeros_v, acc_sh.at[pl.ds(sid * NCHUNK, NCHUNK)])
    pltpu.sync_copy(dst_hbm.at[pl.ds(wid * ECH, ECH)], idx_v)
    pltpu.sync_copy(src_hbm.at[pl.ds(wid * ET, ET)], src_v)
    pltpu.sync_copy(h_hbm, h_v)
    pltpu.sync_copy(deg_hbm, degi_v)
    pltpu.sync_copy(tab_hbm, tab_v)
    plsc.subcore_barrier()

    def chunk_body(j, _):
        for i in range(8):
            t = j * 8 + i
            s_idx = src_v[pl.ds(t * 16, 16)]
            d_idx = idx_v[j, pl.ds(i * 16, 16)]
            hs = plsc.load_gather(h_v, [s_idx])
            dsg = plsc.load_gather(degi_v, [s_idx])
            ddg = plsc.load_gather(degi_v, [d_idx])
            p = jnp.minimum(dsg * ddg, PMAX)
            nrm = plsc.load_gather(tab_v, [p])
            val_v[pl.ds(t * 16, 16)] = hs * nrm
        pltpu.async_copy(val_v.at[pl.ds(j * 128, 128)],
                         acc_sh.at[idx_v.at[j]], sem, add=True)
        return 0
    lax.fori_loop(0, ECH, chunk_body, 0)
    def drain_body(j, _):
        pltpu.make_async_copy(val_v.at[pl.ds(j * 128, 128)],
                              acc_sh.at[idx_v.at[j]], sem).wait()
        return 0
    lax.fori_loop(0, ECH, drain_body, 0)
    plsc.subcore_barrier()

    pltpu.sync_copy(acc_sh.at[pl.ds(sid * NCHUNK, NCHUNK)],
                    out_hbm.at[cid, pl.ds(sid * NCHUNK, NCHUNK)])


def _tc_back_body(s0_ref, s1_ref, hi_ref, s0t_ref, s1t_ref, hit_ref,
                  fun_ref, Wm_ref, bm_ref, lo_ref, fea_ref):
    srow = ((s0_ref[...] + s1_ref[...]) + hi_ref[...])[0]
    scol = ((s0t_ref[...] + s1t_ref[...]) + hit_ref[...])[0]

    ii = lax.broadcasted_iota(jnp.int32, (PER, PER), 0)
    jj = lax.broadcasted_iota(jnp.int32, (PER, PER), 1)
    beats = (scol > srow) | ((scol == srow) & (ii < jj))
    rank = jnp.dot(jnp.ones((1, PER), jnp.bfloat16),
                   beats.astype(jnp.bfloat16),
                   preferred_element_type=jnp.float32)

    pp = lax.broadcasted_iota(jnp.int32, (K, PER), 0).astype(jnp.float32)
    P = (pp == rank).astype(jnp.float32)

    X = fun_ref[...][0] * jnp.tanh(scol)
    fea_st = jnp.dot(P, X, preferred_element_type=jnp.float32,
                     precision=jax.lax.Precision.HIGHEST)
    fea = jnp.dot(fea_st.astype(jnp.bfloat16), Wm_ref[...].astype(jnp.bfloat16),
                  preferred_element_type=jnp.float32) + bm_ref[...]
    fea_ref[...] = fea[None]
    lo_ref[...] = jax.nn.sigmoid(fea)[None]


def _tc_back(s0, s1, hi, Funrep, W_mlp1, b_mlp1):
    row = pl.BlockSpec((1, 1, PER), lambda i: (i, 0, 0))
    col = pl.BlockSpec((1, PER, 1), lambda i: (i, 0, 0))
    full = lambda shape: pl.BlockSpec(shape, lambda i: (0, 0))
    r3 = lambda x: x.reshape(G, 1, PER)
    c3 = lambda x: x.reshape(G, PER, 1)
    return pl.pallas_call(
        _tc_back_body,
        grid=(G,),
        in_specs=[row, row, row, col, col, col,
                  pl.BlockSpec((1, PER, D3), lambda i: (i, 0, 0)),
                  full((D3, MLP1_OUT)), full((1, MLP1_OUT))],
        out_specs=[pl.BlockSpec((1, K, MLP1_OUT), lambda i: (i, 0, 0)),
                   pl.BlockSpec((1, K, MLP1_OUT), lambda i: (i, 0, 0))],
        out_shape=[
            jax.ShapeDtypeStruct((G, K, MLP1_OUT), jnp.float32),
            jax.ShapeDtypeStruct((G, K, MLP1_OUT), jnp.float32),
        ],
    )(r3(s0), r3(s1), r3(hi), c3(s0), c3(s1), c3(hi),
      Funrep.reshape(G, PER, D3), W_mlp1, b_mlp1.reshape(1, MLP1_OUT))


@functools.lru_cache(maxsize=None)
def _sc_kernels():
    mesh = plsc.VectorSubcoreMesh(core_axis_name="c", subcore_axis_name="s")
    params = pltpu.CompilerParams(needs_layout_passes=False)
    sc_deg = pl.kernel(
        _sc_deg_body, mesh=mesh, compiler_params=params,
        out_type=jax.ShapeDtypeStruct((2, NP), jnp.float32),
        scratch_types=[
            pltpu.VMEM((ECH, 128), jnp.int32),
            pltpu.VMEM((128,), jnp.float32),
            pltpu.VMEM((NCHUNK,), jnp.float32),
            pltpu.SemaphoreType.DMA,
            pltpu.VMEM_SHARED((NP,), jnp.float32),
        ],
    )
    sc_msg = pl.kernel(
        _sc_msg_body, mesh=mesh, compiler_params=params,
        out_type=jax.ShapeDtypeStruct((2, NP), jnp.float32),
        scratch_types=[
            pltpu.VMEM((ECH, 128), jnp.int32),
            pltpu.VMEM((ET,), jnp.int32),
            pltpu.VMEM((ET,), jnp.float32),
            pltpu.VMEM((NP,), jnp.float32),
            pltpu.VMEM((NP,), jnp.int32),
            pltpu.VMEM((TABN,), jnp.float32),
            pltpu.VMEM((NCHUNK,), jnp.float32),
            pltpu.SemaphoreType.DMA,
            pltpu.VMEM_SHARED((NP,), jnp.float32),
        ],
    )
    return sc_deg, sc_msg


def kernel(Fea_BP, fea_CC, fea_MF, edge_index, batch, W_bp, b_bp, W_bp1,
           b_bp1, W_att, b_att, v_att, W_score, b_score, W_mlp1, b_mlp1):
    Funrep, h = _tc_front(Fea_BP, fea_CC, fea_MF, W_bp, b_bp, W_bp1, b_bp1,
                          W_att, b_att, v_att, W_score, b_score)

    E_ = edge_index.shape[1]
    fill = jnp.full((EP - E_,), N, dtype=jnp.int32)
    src = jnp.concatenate([edge_index[0], fill])
    dst2d = jnp.concatenate([edge_index[1], fill]).reshape(EP // 128, 128)

    sc_deg, sc_msg = _sc_kernels()
    deg_p = sc_deg(dst2d)
    h_pad = jnp.pad(h[:, 0], (0, NP - N))
    hi, deg_i = _tc_mid(deg_p, h_pad)
    tab = 1.0 / jnp.sqrt(jnp.maximum(jnp.arange(TABN, dtype=jnp.float32), 1.0))
    S_p = sc_msg(src, dst2d, h_pad, deg_i.reshape(NP), tab)

    s0 = S_p[0, :N].reshape(G, PER)
    s1 = S_p[1, :N].reshape(G, PER)
    hi_n = hi.reshape(NP)[:N].reshape(G, PER)

    lo, fea_LO = _tc_back(s0, s1, hi_n, Funrep, W_mlp1, b_mlp1)
    return (lo.reshape(G * K, MLP1_OUT), fea_LO.reshape(G * K, MLP1_OUT), Funrep)


# speedup vs baseline: 1.1253x; 1.0062x over previous
import functools

import jax
import jax.numpy as jnp
from jax import lax
from jax.experimental import pallas as pl
from jax.experimental.pallas import tpu as pltpu
from jax.experimental.pallas import tpu_sc as plsc

N = 10000
G = 20
PER = 500
D1, D2, D3 = 256, 256, 128
K = 250
MLP1_OUT = 512

NP = 10240
NCHUNK = NP // 16
EP = 163840
ET = EP // 32
ECH = ET // 128


def _tc_front_body(bp_ref, cc_ref, mf_ref, Wbp_ref, bbp_ref, Wbp1_ref,
                   bbp1_ref, Watt_ref, batt_ref, vatt_ref, Wsc_ref, bsc_ref,
                   fun_ref, h_ref):
    Wbp = Wbp_ref[...]
    bbp = bbp_ref[...]
    Wbp1 = Wbp1_ref[...]
    bbp1 = bbp1_ref[...]
    Watt = Watt_ref[...]
    batt = batt_ref[...]
    vatt = vatt_ref[...]

    bf = lambda x: x.astype(jnp.bfloat16)

    def dot(a, b):
        return jnp.dot(bf(a), bf(b), preferred_element_type=jnp.float32)

    def proj(x):
        h1 = jnp.maximum(dot(x, Wbp) + bbp, 0.0)
        return jnp.maximum(dot(h1, Wbp1) + bbp1, 0.0)

    def att_logit(h2):
        t = jnp.tanh(dot(h2, Watt) + batt)
        return dot(t, vatt)

    h_bp = proj(bp_ref[...])
    h_cc = proj(cc_ref[...])
    h_mf = proj(mf_ref[...])

    e0 = att_logit(h_bp)
    e1 = att_logit(h_cc)
    e2 = att_logit(h_mf)
    m = jnp.maximum(jnp.maximum(e0, e1), e2)
    u0 = jnp.exp(e0 - m)
    u1 = jnp.exp(e1 - m)
    u2 = jnp.exp(e2 - m)
    denom = u0 + u1 + u2
    a0 = u0 / denom
    a1 = u1 / denom
    a2 = u2 / denom
    fun = a0 * h_bp + a1 * h_cc + a2 * h_mf
    fun_ref[...] = fun
    h_ref[...] = dot(fun, Wsc_ref[...]) + bsc_ref[...]


def _tc_front(Fea_BP, fea_CC, fea_MF, W_bp, b_bp, W_bp1, b_bp1, W_att, b_att,
              v_att, W_score, b_score):
    BR = 1000
    full = lambda shape: pl.BlockSpec(shape, lambda i: (0, 0))
    return pl.pallas_call(
        _tc_front_body,
        grid=(N // BR,),
        in_specs=[
            pl.BlockSpec((BR, D1), lambda i: (i, 0)),
            pl.BlockSpec((BR, D1), lambda i: (i, 0)),
            pl.BlockSpec((BR, D1), lambda i: (i, 0)),
            full((D1, D2)), full((1, D2)),
            full((D2, D3)), full((1, D3)),
            full((D3, D3)), full((1, D3)), full((D3, 1)),
            full((D3, 1)), full((1, 1)),
        ],
        out_specs=[
            pl.BlockSpec((BR, D3), lambda i: (i, 0)),
            pl.BlockSpec((BR, 1), lambda i: (i, 0)),
        ],
        out_shape=[
            jax.ShapeDtypeStruct((N, D3), jnp.float32),
            jax.ShapeDtypeStruct((N, 1), jnp.float32),
        ],
    )(Fea_BP, fea_CC, fea_MF, W_bp, b_bp.reshape(1, D2), W_bp1,
      b_bp1.reshape(1, D3), W_att, b_att.reshape(1, D3), v_att.reshape(D3, 1),
      W_score, b_score.reshape(1, 1))


PMAX = 16384
TABN = PMAX + 128


def _tc_mid_body(d0_ref, d1_ref, h_ref, hi_ref, di_ref):
    deg = d0_ref[...] + d1_ref[...] + 1.0
    deg = jnp.maximum(deg, 1.0)
    hi_ref[...] = h_ref[...] * (1.0 / deg)
    di_ref[...] = deg.astype(jnp.int32)


def _tc_mid(deg_p, h_pad):
    full = pl.BlockSpec((NP // 128, 128), lambda: (0, 0))
    return pl.pallas_call(
        _tc_mid_body,
        in_specs=[full, full, full],
        out_specs=[full, full],
        out_shape=[
            jax.ShapeDtypeStruct((NP // 128, 128), jnp.float32),
            jax.ShapeDtypeStruct((NP // 128, 128), jnp.int32),
        ],
    )(deg_p[0].reshape(NP // 128, 128), deg_p[1].reshape(NP // 128, 128),
      h_pad.reshape(NP // 128, 128))


def _sc_deg_body(dst_hbm, out_hbm, idx_v, ones_v, zeros_v, sem, acc_sh):
    cid = lax.axis_index("c")
    sid = lax.axis_index("s")
    wid = sid * 2 + cid

    zero16 = jnp.zeros((16,), jnp.float32)
    one16 = jnp.ones((16,), jnp.float32)

    def init_body(i, _):
        zeros_v[pl.ds(i * 16, 16)] = zero16
        return 0
    lax.fori_loop(0, NCHUNK // 16, init_body, 0)
    for i in range(8):
        ones_v[pl.ds(i * 16, 16)] = one16

    pltpu.sync_copy(zeros_v, acc_sh.at[pl.ds(sid * NCHUNK, NCHUNK)])
    pltpu.sync_copy(dst_hbm.at[pl.ds(wid * ECH, ECH)], idx_v)
    plsc.subcore_barrier()

    def scat_body(j, _):
        pltpu.async_copy(ones_v, acc_sh.at[idx_v.at[j]], sem, add=True)
        return 0
    lax.fori_loop(0, ECH, scat_body, 0)
    def drain_body(j, _):
        pltpu.make_async_copy(ones_v, acc_sh.at[idx_v.at[j]], sem).wait()
        return 0
    lax.fori_loop(0, ECH, drain_body, 0)
    plsc.subcore_barrier()

    pltpu.sync_copy(acc_sh.at[pl.ds(sid * NCHUNK, NCHUNK)],
                    out_hbm.at[cid, pl.ds(sid * NCHUNK, NCHUNK)])


def _sc_msg_body(src_hbm, dst_hbm, h_hbm, deg_hbm, tab_hbm, out_hbm,
                 idx_v, src_v, val_v, h_v, degi_v, tab_v, zeros_v, sem, acc_sh):
    cid = lax.axis_index("c")
    sid = lax.axis_index("s")
    wid = sid * 2 + cid

    zero16 = jnp.zeros((16,), jnp.float32)

    def init_body(i, _):
        zeros_v[pl.ds(i * 16, 16)] = zero16
        return 0
    lax.fori_loop(0, NCHUNK // 16, init_body, 0)

    pltpu.sync_copy(zeros_v, acc_sh.at[pl.ds(sid * NCHUNK, NCHUNK)])
    pltpu.sync_copy(dst_hbm.at[pl.ds(wid * ECH, ECH)], idx_v)
    pltpu.sync_copy(src_hbm.at[pl.ds(wid * ET, ET)], src_v)
    pltpu.sync_copy(h_hbm, h_v)
    pltpu.sync_copy(deg_hbm, degi_v)
    pltpu.sync_copy(tab_hbm, tab_v)
    plsc.subcore_barrier()

    def chunk_body(j, _):
        for i in range(8):
            t = j * 8 + i
            s_idx = src_v[pl.ds(t * 16, 16)]
            d_idx = idx_v[j, pl.ds(i * 16, 16)]
            hs = plsc.load_gather(h_v, [s_idx])
            dsg = plsc.load_gather(degi_v, [s_idx])
            ddg = plsc.load_gather(degi_v, [d_idx])
            p = jnp.minimum(dsg * ddg, PMAX)
            nrm = plsc.load_gather(tab_v, [p])
            val_v[pl.ds(t * 16, 16)] = hs * nrm
        pltpu.async_copy(val_v.at[pl.ds(j * 128, 128)],
                         acc_sh.at[idx_v.at[j]], sem, add=True)
        return 0
    lax.fori_loop(0, ECH, chunk_body, 0)
    def drain_body(j, _):
        pltpu.make_async_copy(val_v.at[pl.ds(j * 128, 128)],
                              acc_sh.at[idx_v.at[j]], sem).wait()
        return 0
    lax.fori_loop(0, ECH, drain_body, 0)
    plsc.subcore_barrier()

    pltpu.sync_copy(acc_sh.at[pl.ds(sid * NCHUNK, NCHUNK)],
                    out_hbm.at[cid, pl.ds(sid * NCHUNK, NCHUNK)])


def _tc_back_body(s0_ref, s1_ref, hi_ref, s0t_ref, s1t_ref, hit_ref,
                  fun_ref, Wm_ref, bm_ref, lo_ref, fea_ref):
    srow = ((s0_ref[...] + s1_ref[...]) + hi_ref[...])[0]
    scol = ((s0t_ref[...] + s1t_ref[...]) + hit_ref[...])[0]

    ii = lax.broadcasted_iota(jnp.int32, (PER, PER), 0)
    jj = lax.broadcasted_iota(jnp.int32, (PER, PER), 1)
    beats = (scol > srow) | ((scol == srow) & (ii < jj))
    rank = jnp.sum(beats.astype(jnp.float32), axis=0, keepdims=True)

    pp = lax.broadcasted_iota(jnp.int32, (K, PER), 0).astype(jnp.float32)
    P = (pp == rank).astype(jnp.float32)

    X = fun_ref[...][0] * jnp.tanh(scol)
    fea_st = jnp.dot(P, X, preferred_element_type=jnp.float32,
                     precision=jax.lax.Precision.HIGHEST)
    fea = jnp.dot(fea_st.astype(jnp.bfloat16), Wm_ref[...].astype(jnp.bfloat16),
                  preferred_element_type=jnp.float32) + bm_ref[...]
    fea_ref[...] = fea[None]
    lo_ref[...] = jax.nn.sigmoid(fea)[None]


def _tc_back(s0, s1, hi, Funrep, W_mlp1, b_mlp1):
    row = pl.BlockSpec((1, 1, PER), lambda i: (i, 0, 0))
    col = pl.BlockSpec((1, PER, 1), lambda i: (i, 0, 0))
    full = lambda shape: pl.BlockSpec(shape, lambda i: (0, 0))
    r3 = lambda x: x.reshape(G, 1, PER)
    c3 = lambda x: x.reshape(G, PER, 1)
    return pl.pallas_call(
        _tc_back_body,
        grid=(G,),
        in_specs=[row, row, row, col, col, col,
                  pl.BlockSpec((1, PER, D3), lambda i: (i, 0, 0)),
                  full((D3, MLP1_OUT)), full((1, MLP1_OUT))],
        out_specs=[pl.BlockSpec((1, K, MLP1_OUT), lambda i: (i, 0, 0)),
                   pl.BlockSpec((1, K, MLP1_OUT), lambda i: (i, 0, 0))],
        out_shape=[
            jax.ShapeDtypeStruct((G, K, MLP1_OUT), jnp.float32),
            jax.ShapeDtypeStruct((G, K, MLP1_OUT), jnp.float32),
        ],
    )(r3(s0), r3(s1), r3(hi), c3(s0), c3(s1), c3(hi),
      Funrep.reshape(G, PER, D3), W_mlp1, b_mlp1.reshape(1, MLP1_OUT))


@functools.lru_cache(maxsize=None)
def _sc_kernels():
    mesh = plsc.VectorSubcoreMesh(core_axis_name="c", subcore_axis_name="s")
    params = pltpu.CompilerParams(needs_layout_passes=False)
    sc_deg = pl.kernel(
        _sc_deg_body, mesh=mesh, compiler_params=params,
        out_type=jax.ShapeDtypeStruct((2, NP), jnp.float32),
        scratch_types=[
            pltpu.VMEM((ECH, 128), jnp.int32),
            pltpu.VMEM((128,), jnp.float32),
            pltpu.VMEM((NCHUNK,), jnp.float32),
            pltpu.SemaphoreType.DMA,
            pltpu.VMEM_SHARED((NP,), jnp.float32),
        ],
    )
    sc_msg = pl.kernel(
        _sc_msg_body, mesh=mesh, compiler_params=params,
        out_type=jax.ShapeDtypeStruct((2, NP), jnp.float32),
        scratch_types=[
            pltpu.VMEM((ECH, 128), jnp.int32),
            pltpu.VMEM((ET,), jnp.int32),
            pltpu.VMEM((ET,), jnp.float32),
            pltpu.VMEM((NP,), jnp.float32),
            pltpu.VMEM((NP,), jnp.int32),
            pltpu.VMEM((TABN,), jnp.float32),
            pltpu.VMEM((NCHUNK,), jnp.float32),
            pltpu.SemaphoreType.DMA,
            pltpu.VMEM_SHARED((NP,), jnp.float32),
        ],
    )
    return sc_deg, sc_msg


def kernel(Fea_BP, fea_CC, fea_MF, edge_index, batch, W_bp, b_bp, W_bp1,
           b_bp1, W_att, b_att, v_att, W_score, b_score, W_mlp1, b_mlp1):
    Funrep, h = _tc_front(Fea_BP, fea_CC, fea_MF, W_bp, b_bp, W_bp1, b_bp1,
                          W_att, b_att, v_att, W_score, b_score)

    E_ = edge_index.shape[1]
    fill = jnp.full((EP - E_,), N, dtype=jnp.int32)
    src = jnp.concatenate([edge_index[0], fill])
    dst2d = jnp.concatenate([edge_index[1], fill]).reshape(EP // 128, 128)

    sc_deg, sc_msg = _sc_kernels()
    deg_p = sc_deg(dst2d)
    h_pad = jnp.pad(h[:, 0], (0, NP - N))
    hi, deg_i = _tc_mid(deg_p, h_pad)
    tab = 1.0 / jnp.sqrt(jnp.maximum(jnp.arange(TABN, dtype=jnp.float32), 1.0))
    S_p = sc_msg(src, dst2d, h_pad, deg_i.reshape(NP), tab)

    s0 = S_p[0, :N].reshape(G, PER)
    s1 = S_p[1, :N].reshape(G, PER)
    hi_n = hi.reshape(NP)[:N].reshape(G, PER)

    lo, fea_LO = _tc_back(s0, s1, hi_n, Funrep, W_mlp1, b_mlp1)
    return (lo.reshape(G * K, MLP1_OUT), fea_LO.reshape(G * K, MLP1_OUT), Funrep)


# speedup vs baseline: 1.1294x; 1.0036x over previous
import functools

import jax
import jax.numpy as jnp
from jax import lax
from jax.experimental import pallas as pl
from jax.experimental.pallas import tpu as pltpu
from jax.experimental.pallas import tpu_sc as plsc

N = 10000
G = 20
PER = 500
D1, D2, D3 = 256, 256, 128
K = 250
MLP1_OUT = 512

NP = 10240
NCHUNK = NP // 16
EP = 163840
ET = EP // 32
ECH = ET // 128


def _tc_front_body(bp_ref, cc_ref, mf_ref, Wbp_ref, bbp_ref, Wbp1_ref,
                   bbp1_ref, Watt_ref, batt_ref, vatt_ref, Wsc_ref, bsc_ref,
                   fun_ref, h_ref):
    Wbp = Wbp_ref[...]
    bbp = bbp_ref[...]
    Wbp1 = Wbp1_ref[...]
    bbp1 = bbp1_ref[...]
    Watt = Watt_ref[...]
    batt = batt_ref[...]
    vatt = vatt_ref[...]

    bf = lambda x: x.astype(jnp.bfloat16)

    def dot(a, b):
        return jnp.dot(bf(a), bf(b), preferred_element_type=jnp.float32)

    def proj(x):
        h1 = jnp.maximum(dot(x, Wbp) + bbp, 0.0)
        return jnp.maximum(dot(h1, Wbp1) + bbp1, 0.0)

    def att_logit(h2):
        t = jnp.tanh(dot(h2, Watt) + batt)
        return dot(t, vatt)

    h_bp = proj(bp_ref[...])
    h_cc = proj(cc_ref[...])
    h_mf = proj(mf_ref[...])

    e0 = att_logit(h_bp)
    e1 = att_logit(h_cc)
    e2 = att_logit(h_mf)
    m = jnp.maximum(jnp.maximum(e0, e1), e2)
    u0 = jnp.exp(e0 - m)
    u1 = jnp.exp(e1 - m)
    u2 = jnp.exp(e2 - m)
    denom = u0 + u1 + u2
    a0 = u0 / denom
    a1 = u1 / denom
    a2 = u2 / denom
    fun = a0 * h_bp + a1 * h_cc + a2 * h_mf
    fun_ref[...] = fun
    h_ref[...] = dot(fun, Wsc_ref[...]) + bsc_ref[...]


def _tc_front(Fea_BP, fea_CC, fea_MF, W_bp, b_bp, W_bp1, b_bp1, W_att, b_att,
              v_att, W_score, b_score):
    BR = 1000
    full = lambda shape: pl.BlockSpec(shape, lambda i: (0, 0))
    return pl.pallas_call(
        _tc_front_body,
        grid=(N // BR,),
        in_specs=[
            pl.BlockSpec((BR, D1), lambda i: (i, 0)),
            pl.BlockSpec((BR, D1), lambda i: (i, 0)),
            pl.BlockSpec((BR, D1), lambda i: (i, 0)),
            full((D1, D2)), full((1, D2)),
            full((D2, D3)), full((1, D3)),
            full((D3, D3)), full((1, D3)), full((D3, 1)),
            full((D3, 1)), full((1, 1)),
        ],
        out_specs=[
            pl.BlockSpec((BR, D3), lambda i: (i, 0)),
            pl.BlockSpec((BR, 1), lambda i: (i, 0)),
        ],
        out_shape=[
            jax.ShapeDtypeStruct((N, D3), jnp.float32),
            jax.ShapeDtypeStruct((N, 1), jnp.float32),
        ],
    )(Fea_BP, fea_CC, fea_MF, W_bp, b_bp.reshape(1, D2), W_bp1,
      b_bp1.reshape(1, D3), W_att, b_att.reshape(1, D3), v_att.reshape(D3, 1),
      W_score, b_score.reshape(1, 1))


PMAX = 16384
TABN = PMAX + 128


ECH2 = 2 * ECH

def _sc_deg_body(dst_hbm, out_hbm, idx_v, ones_v, zeros_v, degi_v, sem, acc_sh):
    cid = lax.axis_index("c")
    sid = lax.axis_index("s")

    @pl.when(cid == 0)
    def _():
        zero16 = jnp.zeros((16,), jnp.float32)
        one16 = jnp.ones((16,), jnp.float32)

        def init_body(i, _):
            zeros_v[pl.ds(i * 16, 16)] = zero16
            return 0
        lax.fori_loop(0, NCHUNK // 16, init_body, 0)
        for i in range(8):
            ones_v[pl.ds(i * 16, 16)] = one16

        pltpu.sync_copy(zeros_v, acc_sh.at[pl.ds(sid * NCHUNK, NCHUNK)])
        pltpu.sync_copy(dst_hbm.at[pl.ds(sid * ECH2, ECH2)], idx_v)
        plsc.subcore_barrier()

        def scat_body(j, _):
            pltpu.async_copy(ones_v, acc_sh.at[idx_v.at[j]], sem, add=True)
            return 0
        lax.fori_loop(0, ECH2, scat_body, 0)
        def drain_body(j, _):
            pltpu.make_async_copy(ones_v, acc_sh.at[idx_v.at[j]], sem).wait()
            return 0
        lax.fori_loop(0, ECH2, drain_body, 0)
        plsc.subcore_barrier()

        pltpu.sync_copy(acc_sh.at[pl.ds(sid * NCHUNK, NCHUNK)], zeros_v)
        def conv_body(i, _):
            d = zeros_v[pl.ds(i * 16, 16)] + 1.0
            degi_v[pl.ds(i * 16, 16)] = d.astype(jnp.int32)
            return 0
        lax.fori_loop(0, NCHUNK // 16, conv_body, 0)
        pltpu.sync_copy(degi_v, out_hbm.at[pl.ds(sid * NCHUNK, NCHUNK)])


def _sc_msg_body(src_hbm, dst_hbm, h_hbm, deg_hbm, tab_hbm, out_hbm,
                 idx_v, src_v, val_v, h_v, degi_v, tab_v, zeros_v, sem, acc_sh):
    cid = lax.axis_index("c")
    sid = lax.axis_index("s")
    wid = sid * 2 + cid

    zero16 = jnp.zeros((16,), jnp.float32)

    def init_body(i, _):
        zeros_v[pl.ds(i * 16, 16)] = zero16
        return 0
    lax.fori_loop(0, NCHUNK // 16, init_body, 0)

    pltpu.sync_copy(zeros_v, acc_sh.at[pl.ds(sid * NCHUNK, NCHUNK)])
    pltpu.sync_copy(dst_hbm.at[pl.ds(wid * ECH, ECH)], idx_v)
    pltpu.sync_copy(src_hbm.at[pl.ds(wid * ET, ET)], src_v)
    pltpu.sync_copy(h_hbm, h_v)
    pltpu.sync_copy(deg_hbm, degi_v)
    pltpu.sync_copy(tab_hbm, tab_v)
    plsc.subcore_barrier()

    def chunk_body(j, _):
        for i in range(8):
            t = j * 8 + i
            s_idx = src_v[pl.ds(t * 16, 16)]
            d_idx = idx_v[j, pl.ds(i * 16, 16)]
            hs = plsc.load_gather(h_v, [s_idx])
            dsg = plsc.load_gather(degi_v, [s_idx])
            ddg = plsc.load_gather(degi_v, [d_idx])
            p = jnp.minimum(dsg * ddg, PMAX)
            nrm = plsc.load_gather(tab_v, [p])
            val_v[pl.ds(t * 16, 16)] = hs * nrm
        pltpu.async_copy(val_v.at[pl.ds(j * 128, 128)],
                         acc_sh.at[idx_v.at[j]], sem, add=True)
        return 0
    lax.fori_loop(0, ECH, chunk_body, 0)
    def drain_body(j, _):
        pltpu.make_async_copy(val_v.at[pl.ds(j * 128, 128)],
                              acc_sh.at[idx_v.at[j]], sem).wait()
        return 0
    lax.fori_loop(0, ECH, drain_body, 0)
    plsc.subcore_barrier()

    pltpu.sync_copy(acc_sh.at[pl.ds(sid * NCHUNK, NCHUNK)],
                    out_hbm.at[cid, pl.ds(sid * NCHUNK, NCHUNK)])


def _tc_back_body(s0_ref, s1_ref, h_ref, d_ref, s0t_ref, s1t_ref, ht_ref,
                  dt_ref, fun_ref, Wm_ref, bm_ref, lo_ref, fea_ref):
    hi_row = h_ref[...] * (1.0 / d_ref[...].astype(jnp.float32))
    hi_col = ht_ref[...] * (1.0 / dt_ref[...].astype(jnp.float32))
    srow = ((s0_ref[...] + s1_ref[...]) + hi_row)[0]
    scol = ((s0t_ref[...] + s1t_ref[...]) + hi_col)[0]

    ii = lax.broadcasted_iota(jnp.int32, (PER, PER), 0)
    jj = lax.broadcasted_iota(jnp.int32, (PER, PER), 1)
    beats = (scol > srow) | ((scol == srow) & (ii < jj))
    rank = jnp.sum(beats.astype(jnp.float32), axis=0, keepdims=True)

    pp = lax.broadcasted_iota(jnp.int32, (K, PER), 0).astype(jnp.float32)
    P = (pp == rank).astype(jnp.float32)

    X = fun_ref[...][0] * jnp.tanh(scol)
    fea_st = jnp.dot(P, X, preferred_element_type=jnp.float32,
                     precision=jax.lax.Precision.HIGHEST)
    fea = jnp.dot(fea_st.astype(jnp.bfloat16), Wm_ref[...].astype(jnp.bfloat16),
                  preferred_element_type=jnp.float32) + bm_ref[...]
    fea_ref[...] = fea[None]
    lo_ref[...] = jax.nn.sigmoid(fea)[None]


def _tc_back(s0, s1, h_n, deg_n, Funrep, W_mlp1, b_mlp1):
    row = pl.BlockSpec((1, 1, PER), lambda i: (i, 0, 0))
    col = pl.BlockSpec((1, PER, 1), lambda i: (i, 0, 0))
    full = lambda shape: pl.BlockSpec(shape, lambda i: (0, 0))
    r3 = lambda x: x.reshape(G, 1, PER)
    c3 = lambda x: x.reshape(G, PER, 1)
    return pl.pallas_call(
        _tc_back_body,
        grid=(G,),
        in_specs=[row, row, row, row, col, col, col, col,
                  pl.BlockSpec((1, PER, D3), lambda i: (i, 0, 0)),
                  full((D3, MLP1_OUT)), full((1, MLP1_OUT))],
        out_specs=[pl.BlockSpec((1, K, MLP1_OUT), lambda i: (i, 0, 0)),
                   pl.BlockSpec((1, K, MLP1_OUT), lambda i: (i, 0, 0))],
        out_shape=[
            jax.ShapeDtypeStruct((G, K, MLP1_OUT), jnp.float32),
            jax.ShapeDtypeStruct((G, K, MLP1_OUT), jnp.float32),
        ],
    )(r3(s0), r3(s1), r3(h_n), r3(deg_n), c3(s0), c3(s1), c3(h_n), c3(deg_n),
      Funrep.reshape(G, PER, D3), W_mlp1, b_mlp1.reshape(1, MLP1_OUT))


@functools.lru_cache(maxsize=None)
def _sc_kernels():
    mesh = plsc.VectorSubcoreMesh(core_axis_name="c", subcore_axis_name="s")
    params = pltpu.CompilerParams(needs_layout_passes=False)
    sc_deg = pl.kernel(
        _sc_deg_body, mesh=mesh, compiler_params=params,
        out_type=jax.ShapeDtypeStruct((NP,), jnp.int32),
        scratch_types=[
            pltpu.VMEM((ECH2, 128), jnp.int32),
            pltpu.VMEM((128,), jnp.float32),
            pltpu.VMEM((NCHUNK,), jnp.float32),
            pltpu.VMEM((NCHUNK,), jnp.int32),
            pltpu.SemaphoreType.DMA,
            pltpu.VMEM_SHARED((NP,), jnp.float32),
        ],
    )
    sc_msg = pl.kernel(
        _sc_msg_body, mesh=mesh, compiler_params=params,
        out_type=jax.ShapeDtypeStruct((2, NP), jnp.float32),
        scratch_types=[
            pltpu.VMEM((ECH, 128), jnp.int32),
            pltpu.VMEM((ET,), jnp.int32),
            pltpu.VMEM((ET,), jnp.float32),
            pltpu.VMEM((NP,), jnp.float32),
            pltpu.VMEM((NP,), jnp.int32),
            pltpu.VMEM((TABN,), jnp.float32),
            pltpu.VMEM((NCHUNK,), jnp.float32),
            pltpu.SemaphoreType.DMA,
            pltpu.VMEM_SHARED((NP,), jnp.float32),
        ],
    )
    return sc_deg, sc_msg


def kernel(Fea_BP, fea_CC, fea_MF, edge_index, batch, W_bp, b_bp, W_bp1,
           b_bp1, W_att, b_att, v_att, W_score, b_score, W_mlp1, b_mlp1):
    Funrep, h = _tc_front(Fea_BP, fea_CC, fea_MF, W_bp, b_bp, W_bp1, b_bp1,
                          W_att, b_att, v_att, W_score, b_score)

    E_ = edge_index.shape[1]
    fill = jnp.full((EP - E_,), N, dtype=jnp.int32)
    src = jnp.concatenate([edge_index[0], fill])
    dst2d = jnp.concatenate([edge_index[1], fill]).reshape(EP // 128, 128)

    sc_deg, sc_msg = _sc_kernels()
    deg_i = sc_deg(dst2d)
    h_pad = jnp.pad(h[:, 0], (0, NP - N))
    tab = 1.0 / jnp.sqrt(jnp.maximum(jnp.arange(TABN, dtype=jnp.float32), 1.0))
    S_p = sc_msg(src, dst2d, h_pad, deg_i, tab)

    s0 = S_p[0, :N].reshape(G, PER)
    s1 = S_p[1, :N].reshape(G, PER)
    h_n = h[:, 0].reshape(G, PER)
    deg_n = deg_i[:N].reshape(G, PER)

    lo, fea_LO = _tc_back(s0, s1, h_n, deg_n, Funrep, W_mlp1, b_mlp1)
    return (lo.reshape(G * K, MLP1_OUT), fea_LO.reshape(G * K, MLP1_OUT), Funrep)


# speedup vs baseline: 1.1363x; 1.0061x over previous
import functools

import jax
import jax.numpy as jnp
from jax import lax
from jax.experimental import pallas as pl
from jax.experimental.pallas import tpu as pltpu
from jax.experimental.pallas import tpu_sc as plsc

N = 10000
G = 20
PER = 500
D1, D2, D3 = 256, 256, 128
K = 250
MLP1_OUT = 512

NP = 10240
NCHUNK = NP // 16
EP = 163840
ET = EP // 32
ECH = ET // 128


def _tc_front_body(bp_ref, cc_ref, mf_ref, Wbp_ref, bbp_ref, Wbp1_ref,
                   bbp1_ref, Watt_ref, batt_ref, vatt_ref, Wsc_ref, bsc_ref,
                   fun_ref, h_ref):
    Wbp = Wbp_ref[...]
    bbp = bbp_ref[...]
    Wbp1 = Wbp1_ref[...]
    bbp1 = bbp1_ref[...]
    Watt = Watt_ref[...]
    batt = batt_ref[...]
    vatt = vatt_ref[...]

    bf = lambda x: x.astype(jnp.bfloat16)

    def dot(a, b):
        return jnp.dot(bf(a), bf(b), preferred_element_type=jnp.float32)

    def proj(x):
        h1 = jnp.maximum(dot(x, Wbp) + bbp, 0.0)
        return jnp.maximum(dot(h1, Wbp1) + bbp1, 0.0)

    def att_logit(h2):
        t = jnp.tanh(dot(h2, Watt) + batt)
        return dot(t, vatt)

    h_bp = proj(bp_ref[...])
    h_cc = proj(cc_ref[...])
    h_mf = proj(mf_ref[...])

    e0 = att_logit(h_bp)
    e1 = att_logit(h_cc)
    e2 = att_logit(h_mf)
    m = jnp.maximum(jnp.maximum(e0, e1), e2)
    u0 = jnp.exp(e0 - m)
    u1 = jnp.exp(e1 - m)
    u2 = jnp.exp(e2 - m)
    denom = u0 + u1 + u2
    a0 = u0 / denom
    a1 = u1 / denom
    a2 = u2 / denom
    fun = a0 * h_bp + a1 * h_cc + a2 * h_mf
    fun_ref[...] = fun
    h_ref[...] = dot(fun, Wsc_ref[...]) + bsc_ref[...]


def _tc_front(Fea_BP, fea_CC, fea_MF, W_bp, b_bp, W_bp1, b_bp1, W_att, b_att,
              v_att, W_score, b_score):
    BR = 2000
    full = lambda shape: pl.BlockSpec(shape, lambda i: (0, 0))
    return pl.pallas_call(
        _tc_front_body,
        grid=(N // BR,),
        in_specs=[
            pl.BlockSpec((BR, D1), lambda i: (i, 0)),
            pl.BlockSpec((BR, D1), lambda i: (i, 0)),
            pl.BlockSpec((BR, D1), lambda i: (i, 0)),
            full((D1, D2)), full((1, D2)),
            full((D2, D3)), full((1, D3)),
            full((D3, D3)), full((1, D3)), full((D3, 1)),
            full((D3, 1)), full((1, 1)),
        ],
        out_specs=[
            pl.BlockSpec((BR, D3), lambda i: (i, 0)),
            pl.BlockSpec((BR, 1), lambda i: (i, 0)),
        ],
        out_shape=[
            jax.ShapeDtypeStruct((N, D3), jnp.float32),
            jax.ShapeDtypeStruct((N, 1), jnp.float32),
        ],
    )(Fea_BP, fea_CC, fea_MF, W_bp, b_bp.reshape(1, D2), W_bp1,
      b_bp1.reshape(1, D3), W_att, b_att.reshape(1, D3), v_att.reshape(D3, 1),
      W_score, b_score.reshape(1, 1))


PMAX = 16384
TABN = PMAX + 128


ECH2 = 2 * ECH

def _sc_deg_body(dst_hbm, out_hbm, idx_v, ones_v, zeros_v, degi_v, sem, acc_sh):
    cid = lax.axis_index("c")
    sid = lax.axis_index("s")

    @pl.when(cid == 0)
    def _():
        zero16 = jnp.zeros((16,), jnp.float32)
        one16 = jnp.ones((16,), jnp.float32)

        def init_body(i, _):
            zeros_v[pl.ds(i * 16, 16)] = zero16
            return 0
        lax.fori_loop(0, NCHUNK // 16, init_body, 0)
        for i in range(8):
            ones_v[pl.ds(i * 16, 16)] = one16

        pltpu.sync_copy(zeros_v, acc_sh.at[pl.ds(sid * NCHUNK, NCHUNK)])
        pltpu.sync_copy(dst_hbm.at[pl.ds(sid * ECH2, ECH2)], idx_v)
        plsc.subcore_barrier()

        def scat_body(j, _):
            pltpu.async_copy(ones_v, acc_sh.at[idx_v.at[j]], sem, add=True)
            return 0
        lax.fori_loop(0, ECH2, scat_body, 0)
        def drain_body(j, _):
            pltpu.make_async_copy(ones_v, acc_sh.at[idx_v.at[j]], sem).wait()
            return 0
        lax.fori_loop(0, ECH2, drain_body, 0)
        plsc.subcore_barrier()

        pltpu.sync_copy(acc_sh.at[pl.ds(sid * NCHUNK, NCHUNK)], zeros_v)
        def conv_body(i, _):
            d = zeros_v[pl.ds(i * 16, 16)] + 1.0
            degi_v[pl.ds(i * 16, 16)] = d.astype(jnp.int32)
            return 0
        lax.fori_loop(0, NCHUNK // 16, conv_body, 0)
        pltpu.sync_copy(degi_v, out_hbm.at[pl.ds(sid * NCHUNK, NCHUNK)])


def _sc_msg_body(src_hbm, dst_hbm, h_hbm, deg_hbm, tab_hbm, out_hbm,
                 idx_v, src_v, val_v, h_v, degi_v, tab_v, zeros_v, sem, acc_sh):
    cid = lax.axis_index("c")
    sid = lax.axis_index("s")
    wid = sid * 2 + cid

    zero16 = jnp.zeros((16,), jnp.float32)

    def init_body(i, _):
        zeros_v[pl.ds(i * 16, 16)] = zero16
        return 0
    lax.fori_loop(0, NCHUNK // 16, init_body, 0)

    pltpu.sync_copy(zeros_v, acc_sh.at[pl.ds(sid * NCHUNK, NCHUNK)])
    pltpu.sync_copy(dst_hbm.at[pl.ds(wid * ECH, ECH)], idx_v)
    pltpu.sync_copy(src_hbm.at[pl.ds(wid * ET, ET)], src_v)
    pltpu.sync_copy(h_hbm, h_v)
    pltpu.sync_copy(deg_hbm, degi_v)
    pltpu.sync_copy(tab_hbm, tab_v)
    plsc.subcore_barrier()

    def chunk_body(j, _):
        for i in range(8):
            t = j * 8 + i
            s_idx = src_v[pl.ds(t * 16, 16)]
            d_idx = idx_v[j, pl.ds(i * 16, 16)]
            hs = plsc.load_gather(h_v, [s_idx])
            dsg = plsc.load_gather(degi_v, [s_idx])
            ddg = plsc.load_gather(degi_v, [d_idx])
            p = jnp.minimum(dsg * ddg, PMAX)
            nrm = plsc.load_gather(tab_v, [p])
            val_v[pl.ds(t * 16, 16)] = hs * nrm
        pltpu.async_copy(val_v.at[pl.ds(j * 128, 128)],
                         acc_sh.at[idx_v.at[j]], sem, add=True)
        return 0
    lax.fori_loop(0, ECH, chunk_body, 0)
    def drain_body(j, _):
        pltpu.make_async_copy(val_v.at[pl.ds(j * 128, 128)],
                              acc_sh.at[idx_v.at[j]], sem).wait()
        return 0
    lax.fori_loop(0, ECH, drain_body, 0)
    plsc.subcore_barrier()

    pltpu.sync_copy(acc_sh.at[pl.ds(sid * NCHUNK, NCHUNK)],
                    out_hbm.at[cid, pl.ds(sid * NCHUNK, NCHUNK)])


def _tc_back_body(s0_ref, s1_ref, h_ref, d_ref, s0t_ref, s1t_ref, ht_ref,
                  dt_ref, fun_ref, Wm_ref, bm_ref, lo_ref, fea_ref):
    hi_row = h_ref[...] * (1.0 / d_ref[...].astype(jnp.float32))
    hi_col = ht_ref[...] * (1.0 / dt_ref[...].astype(jnp.float32))
    srow = ((s0_ref[...] + s1_ref[...]) + hi_row)[0]
    scol = ((s0t_ref[...] + s1t_ref[...]) + hi_col)[0]

    ii = lax.broadcasted_iota(jnp.int32, (PER, PER), 0)
    jj = lax.broadcasted_iota(jnp.int32, (PER, PER), 1)
    beats = (scol > srow) | ((scol == srow) & (ii < jj))
    rank = jnp.sum(beats.astype(jnp.float32), axis=0, keepdims=True)

    pp = lax.broadcasted_iota(jnp.int32, (K, PER), 0).astype(jnp.float32)
    P = (pp == rank).astype(jnp.float32)

    X = fun_ref[...][0] * jnp.tanh(scol)
    fea_st = jnp.dot(P, X, preferred_element_type=jnp.float32,
                     precision=jax.lax.Precision.HIGHEST)
    fea = jnp.dot(fea_st.astype(jnp.bfloat16), Wm_ref[...].astype(jnp.bfloat16),
                  preferred_element_type=jnp.float32) + bm_ref[...]
    fea_ref[...] = fea[None]
    lo_ref[...] = jax.nn.sigmoid(fea)[None]


def _tc_back(s0, s1, h_n, deg_n, Funrep, W_mlp1, b_mlp1):
    row = pl.BlockSpec((1, 1, PER), lambda i: (i, 0, 0))
    col = pl.BlockSpec((1, PER, 1), lambda i: (i, 0, 0))
    full = lambda shape: pl.BlockSpec(shape, lambda i: (0, 0))
    r3 = lambda x: x.reshape(G, 1, PER)
    c3 = lambda x: x.reshape(G, PER, 1)
    return pl.pallas_call(
        _tc_back_body,
        grid=(G,),
        in_specs=[row, row, row, row, col, col, col, col,
                  pl.BlockSpec((1, PER, D3), lambda i: (i, 0, 0)),
                  full((D3, MLP1_OUT)), full((1, MLP1_OUT))],
        out_specs=[pl.BlockSpec((1, K, MLP1_OUT), lambda i: (i, 0, 0)),
                   pl.BlockSpec((1, K, MLP1_OUT), lambda i: (i, 0, 0))],
        out_shape=[
            jax.ShapeDtypeStruct((G, K, MLP1_OUT), jnp.float32),
            jax.ShapeDtypeStruct((G, K, MLP1_OUT), jnp.float32),
        ],
    )(r3(s0), r3(s1), r3(h_n), r3(deg_n), c3(s0), c3(s1), c3(h_n), c3(deg_n),
      Funrep.reshape(G, PER, D3), W_mlp1, b_mlp1.reshape(1, MLP1_OUT))


@functools.lru_cache(maxsize=None)
def _sc_kernels():
    mesh = plsc.VectorSubcoreMesh(core_axis_name="c", subcore_axis_name="s")
    params = pltpu.CompilerParams(needs_layout_passes=False)
    sc_deg = pl.kernel(
        _sc_deg_body, mesh=mesh, compiler_params=params,
        out_type=jax.ShapeDtypeStruct((NP,), jnp.int32),
        scratch_types=[
            pltpu.VMEM((ECH2, 128), jnp.int32),
            pltpu.VMEM((128,), jnp.float32),
            pltpu.VMEM((NCHUNK,), jnp.float32),
            pltpu.VMEM((NCHUNK,), jnp.int32),
            pltpu.SemaphoreType.DMA,
            pltpu.VMEM_SHARED((NP,), jnp.float32),
        ],
    )
    sc_msg = pl.kernel(
        _sc_msg_body, mesh=mesh, compiler_params=params,
        out_type=jax.ShapeDtypeStruct((2, NP), jnp.float32),
        scratch_types=[
            pltpu.VMEM((ECH, 128), jnp.int32),
            pltpu.VMEM((ET,), jnp.int32),
            pltpu.VMEM((ET,), jnp.float32),
            pltpu.VMEM((NP,), jnp.float32),
            pltpu.VMEM((NP,), jnp.int32),
            pltpu.VMEM((TABN,), jnp.float32),
            pltpu.VMEM((NCHUNK,), jnp.float32),
            pltpu.SemaphoreType.DMA,
            pltpu.VMEM_SHARED((NP,), jnp.float32),
        ],
    )
    return sc_deg, sc_msg


def kernel(Fea_BP, fea_CC, fea_MF, edge_index, batch, W_bp, b_bp, W_bp1,
           b_bp1, W_att, b_att, v_att, W_score, b_score, W_mlp1, b_mlp1):
    Funrep, h = _tc_front(Fea_BP, fea_CC, fea_MF, W_bp, b_bp, W_bp1, b_bp1,
                          W_att, b_att, v_att, W_score, b_score)

    E_ = edge_index.shape[1]
    fill = jnp.full((EP - E_,), N, dtype=jnp.int32)
    src = jnp.concatenate([edge_index[0], fill])
    dst2d = jnp.concatenate([edge_index[1], fill]).reshape(EP // 128, 128)

    sc_deg, sc_msg = _sc_kernels()
    deg_i = sc_deg(dst2d)
    h_pad = jnp.pad(h[:, 0], (0, NP - N))
    tab = 1.0 / jnp.sqrt(jnp.maximum(jnp.arange(TABN, dtype=jnp.float32), 1.0))
    S_p = sc_msg(src, dst2d, h_pad, deg_i, tab)

    s0 = S_p[0, :N].reshape(G, PER)
    s1 = S_p[1, :N].reshape(G, PER)
    h_n = h[:, 0].reshape(G, PER)
    deg_n = deg_i[:N].reshape(G, PER)

    lo, fea_LO = _tc_back(s0, s1, h_n, deg_n, Funrep, W_mlp1, b_mlp1)
    return (lo.reshape(G * K, MLP1_OUT), fea_LO.reshape(G * K, MLP1_OUT), Funrep)
